# Initial kernel scaffold; baseline (speedup 1.0000x reference)
import jax
import jax.numpy as jnp
from jax.experimental import pallas as pl


def kernel(x, edge_index, W0, b0, prelu0, W1, b1, prelu1, Wout, bout):
    raise NotImplementedError("write your pallas kernel here")


# speedup vs baseline: 5.8456x; 5.8456x over previous
import functools

import jax
import jax.numpy as jnp
from jax import lax
from jax.experimental import pallas as pl
from jax.experimental.pallas import tpu as pltpu
from jax.experimental.pallas import tpu_sc as plsc

N = 10000
D = 128
E = 320000
NPAD = 10240
NC, NS = 2, 16
NW = NC * NS
CHUNK = 128
CHUNKS_PER_W = 79
EPW = CHUNKS_PER_W * CHUNK
EPAD = NW * EPW
RPW = NPAD // NS
BLK = 1280
GRID = NPAD // BLK

_PREC = jax.lax.Precision.HIGHEST


def _zero2d(ref, rows, cols):
    def row(r, _):
        def col(j, _):
            ref[r, pl.ds(j * 16, 16)] = jnp.zeros((16,), jnp.float32)
            return 0
        return lax.fori_loop(0, cols // 16, col, 0)
    lax.fori_loop(0, rows, row, 0)


@functools.cache
def _build_sc_prop():
    mesh = plsc.VectorSubcoreMesh(core_axis_name="c", subcore_axis_name="s")

    @functools.partial(
        pl.kernel,
        out_type=jax.ShapeDtypeStruct((NC * NPAD, D), jnp.float32),
        mesh=mesh,
        scratch_types=[
            pltpu.VMEM((CHUNK,), jnp.int32),
            pltpu.VMEM((CHUNK,), jnp.int32),
            pltpu.VMEM((CHUNK, D), jnp.float32),
            pltpu.VMEM_SHARED((NPAD, D), jnp.float32),
            pltpu.SemaphoreType.DMA,
        ],
    )
    def sc_prop(u_hbm, src_hbm, dst_hbm, out_hbm, src_v, dst_v, rows_v,
                acc_sh, sem):
        c = lax.axis_index("c")
        s = lax.axis_index("s")
        wid = c * NS + s
        _zero2d(rows_v, CHUNK, D)
        for t in range(RPW // CHUNK):
            pltpu.sync_copy(rows_v, acc_sh.at[pl.ds(s * RPW + t * CHUNK, CHUNK)])
        plsc.subcore_barrier()

        def body(i, _):
            base = wid * EPW + i * CHUNK
            pltpu.sync_copy(src_hbm.at[pl.ds(base, CHUNK)], src_v)
            pltpu.sync_copy(dst_hbm.at[pl.ds(base, CHUNK)], dst_v)
            pltpu.async_copy(u_hbm.at[src_v], rows_v, sem).wait()
            pltpu.sync_copy(rows_v, acc_sh.at[dst_v], add=True)
            return 0
        lax.fori_loop(0, CHUNKS_PER_W, body, 0)
        plsc.subcore_barrier()
        pltpu.sync_copy(acc_sh.at[pl.ds(s * RPW, RPW)],
                        out_hbm.at[pl.ds(c * NPAD + s * RPW, RPW)])

    return sc_prop


def _sc_prop(u, src_pad, dst_pad):
    return _build_sc_prop()(u, src_pad, dst_pad)


@functools.cache
def _build_sc_deg():
    mesh = plsc.VectorSubcoreMesh(core_axis_name="c", subcore_axis_name="s")

    @functools.partial(
        pl.kernel,
        out_type=jax.ShapeDtypeStruct((NC * NPAD, D), jnp.float32),
        mesh=mesh,
        scratch_types=[
            pltpu.VMEM((CHUNK,), jnp.int32),
            pltpu.VMEM((CHUNK, D), jnp.float32),
            pltpu.VMEM((CHUNK, D), jnp.float32),
            pltpu.VMEM_SHARED((NPAD, D), jnp.float32),
            pltpu.SemaphoreType.DMA,
        ],
    )
    def sc_deg(dst_hbm, out_hbm, dst_v, ones_v, zero_v, acc_sh, sem):
        c = lax.axis_index("c")
        s = lax.axis_index("s")
        wid = c * NS + s
        _zero2d(zero_v, CHUNK, D)

        def fill(r, _):
            def col(j, _):
                ones_v[r, pl.ds(j * 16, 16)] = jnp.ones((16,), jnp.float32)
                return 0
            return lax.fori_loop(0, D // 16, col, 0)
        lax.fori_loop(0, CHUNK, fill, 0)
        for t in range(RPW // CHUNK):
            pltpu.sync_copy(zero_v, acc_sh.at[pl.ds(s * RPW + t * CHUNK, CHUNK)])
        plsc.subcore_barrier()

        def body(i, _):
            base = wid * EPW + i * CHUNK
            pltpu.sync_copy(dst_hbm.at[pl.ds(base, CHUNK)], dst_v)
            pltpu.sync_copy(ones_v, acc_sh.at[dst_v], add=True)
            return 0
        lax.fori_loop(0, CHUNKS_PER_W, body, 0)
        plsc.subcore_barrier()
        pltpu.sync_copy(acc_sh.at[pl.ds(s * RPW, RPW)],
                        out_hbm.at[pl.ds(c * NPAD + s * RPW, RPW)])

    return sc_deg


def _sc_deg(dst_pad):
    return _build_sc_deg()(dst_pad)


def _dis_body(deg_ref, dis_ref):
    deg = deg_ref[0, :, 0:1] + deg_ref[1, :, 0:1]
    dis_ref[...] = jnp.where(deg > 0, lax.rsqrt(deg), 0.0)


def _tc_dis(deg_p):
    return pl.pallas_call(
        _dis_body,
        out_shape=jax.ShapeDtypeStruct((NPAD, 1), jnp.float32),
    )(deg_p.reshape(NC, NPAD, D))


def _start_body(x_ref, dis_ref, w_ref, u_ref, acc_ref):
    x = x_ref[...]
    u_ref[...] = dis_ref[...] * x
    acc_ref[...] = lax.dot_general(x, w_ref[...], (((1,), (1,)), ((), ())),
                                   precision=_PREC)


def _tc_start(x_pad, dis, w0):
    return pl.pallas_call(
        _start_body,
        grid=(GRID,),
        in_specs=[
            pl.BlockSpec((BLK, D), lambda i: (i, 0)),
            pl.BlockSpec((BLK, 1), lambda i: (i, 0)),
            pl.BlockSpec((D, D), lambda i: (0, 0)),
        ],
        out_specs=[
            pl.BlockSpec((BLK, D), lambda i: (i, 0)),
            pl.BlockSpec((BLK, D), lambda i: (i, 0)),
        ],
        out_shape=[
            jax.ShapeDtypeStruct((NPAD, D), jnp.float32),
            jax.ShapeDtypeStruct((NPAD, D), jnp.float32),
        ],
    )(x_pad, dis, w0)


def _mid_body(p_ref, dis_ref, acc_ref, w_ref, accout_ref, unext_ref):
    v = p_ref[0] + p_ref[1]
    dis = dis_ref[...]
    h = dis * v
    accout_ref[...] = acc_ref[...] + lax.dot_general(
        h, w_ref[...], (((1,), (1,)), ((), ())), precision=_PREC)
    unext_ref[...] = dis * h


def _tc_mid(p, dis, acc, wk):
    return pl.pallas_call(
        _mid_body,
        grid=(GRID,),
        in_specs=[
            pl.BlockSpec((NC, BLK, D), lambda i: (0, i, 0)),
            pl.BlockSpec((BLK, 1), lambda i: (i, 0)),
            pl.BlockSpec((BLK, D), lambda i: (i, 0)),
            pl.BlockSpec((D, D), lambda i: (0, 0)),
        ],
        out_specs=[
            pl.BlockSpec((BLK, D), lambda i: (i, 0)),
            pl.BlockSpec((BLK, D), lambda i: (i, 0)),
        ],
        out_shape=[
            jax.ShapeDtypeStruct((NPAD, D), jnp.float32),
            jax.ShapeDtypeStruct((NPAD, D), jnp.float32),
        ],
    )(p.reshape(NC, NPAD, D), dis, acc, wk)


def _end_body(p_ref, dis_ref, acc_ref, w_ref, b_ref, a_ref, wn_ref,
              unext_ref, accnext_ref):
    i = pl.program_id(0)
    v = p_ref[0] + p_ref[1]
    dis = dis_ref[...]
    h = dis * v
    rows = acc_ref[...] + lax.dot_general(
        h, w_ref[...], (((1,), (1,)), ((), ())), precision=_PREC) + b_ref[...]
    a = a_ref[0, 0]
    g = jnp.where(rows > 0, rows, a * rows)
    rid = i * BLK + lax.broadcasted_iota(jnp.int32, (BLK, D), 0)
    g = jnp.where(rid < N, g, 0.0)
    unext_ref[...] = dis * g
    accnext_ref[...] = lax.dot_general(
        g, wn_ref[...], (((1,), (1,)), ((), ())), precision=_PREC)


def _tc_end(p, dis, acc, wk, b, a, wnext):
    return pl.pallas_call(
        _end_body,
        grid=(GRID,),
        in_specs=[
            pl.BlockSpec((NC, BLK, D), lambda i: (0, i, 0)),
            pl.BlockSpec((BLK, 1), lambda i: (i, 0)),
            pl.BlockSpec((BLK, D), lambda i: (i, 0)),
            pl.BlockSpec((D, D), lambda i: (0, 0)),
            pl.BlockSpec((1, D), lambda i: (0, 0)),
            pl.BlockSpec(memory_space=pltpu.SMEM),
            pl.BlockSpec((D, D), lambda i: (0, 0)),
        ],
        out_specs=[
            pl.BlockSpec((BLK, D), lambda i: (i, 0)),
            pl.BlockSpec((BLK, D), lambda i: (i, 0)),
        ],
        out_shape=[
            jax.ShapeDtypeStruct((NPAD, D), jnp.float32),
            jax.ShapeDtypeStruct((NPAD, D), jnp.float32),
        ],
    )(p.reshape(NC, NPAD, D), dis, acc, wk, b.reshape(1, D),
      a.reshape(1, 1), wnext)


def _final_body(p_ref, dis_ref, acc_ref, w_ref, b_ref, a_ref, wout_ref,
                bout_ref, out_ref):
    i = pl.program_id(0)
    v = p_ref[0] + p_ref[1]
    h = dis_ref[...] * v
    rows = acc_ref[...] + lax.dot_general(
        h, w_ref[...], (((1,), (1,)), ((), ())), precision=_PREC) + b_ref[...]
    a = a_ref[0, 0]
    g = jnp.where(rows > 0, rows, a * rows)
    rid = i * BLK + lax.broadcasted_iota(jnp.int32, (BLK, D), 0)
    g = jnp.where(rid < N, g, 0.0)
    part = jnp.sum(g * wout_ref[...])

    @pl.when(i == 0)
    def _():
        out_ref[0, 0] = bout_ref[0, 0] + part

    @pl.when(i > 0)
    def _():
        out_ref[0, 0] += part


def _tc_final(p, dis, acc, wk, b, a, wout, bout):
    return pl.pallas_call(
        _final_body,
        grid=(GRID,),
        in_specs=[
            pl.BlockSpec((NC, BLK, D), lambda i: (0, i, 0)),
            pl.BlockSpec((BLK, 1), lambda i: (i, 0)),
            pl.BlockSpec((BLK, D), lambda i: (i, 0)),
            pl.BlockSpec((D, D), lambda i: (0, 0)),
            pl.BlockSpec((1, D), lambda i: (0, 0)),
            pl.BlockSpec(memory_space=pltpu.SMEM),
            pl.BlockSpec((1, D), lambda i: (0, 0)),
            pl.BlockSpec(memory_space=pltpu.SMEM),
        ],
        out_specs=pl.BlockSpec(memory_space=pltpu.SMEM),
        out_shape=jax.ShapeDtypeStruct((1, 1), jnp.float32),
    )(p.reshape(NC, NPAD, D), dis, acc, wk, b.reshape(1, D),
      a.reshape(1, 1), wout, bout.reshape(1, 1))


def kernel(x, edge_index, W0, b0, prelu0, W1, b1, prelu1, Wout, bout):
    src = edge_index[0].astype(jnp.int32)
    dst = edge_index[1].astype(jnp.int32)
    pad = jnp.full((EPAD - E,), N, dtype=jnp.int32)
    src_pad = jnp.concatenate([src, pad])
    dst_pad = jnp.concatenate([dst, pad])
    x_pad = jnp.zeros((NPAD, D), jnp.float32).at[:N].set(x)

    deg_p = _sc_deg(dst_pad)
    dis = _tc_dis(deg_p)

    u, acc = _tc_start(x_pad, dis, W0[0])
    for k in (1, 2):
        p = _sc_prop(u, src_pad, dst_pad)
        acc, u = _tc_mid(p, dis, acc, W0[k])
    p = _sc_prop(u, src_pad, dst_pad)
    u, acc = _tc_end(p, dis, acc, W0[3], b0, prelu0, W1[0])

    for k in (1, 2):
        p = _sc_prop(u, src_pad, dst_pad)
        acc, u = _tc_mid(p, dis, acc, W1[k])
    p = _sc_prop(u, src_pad, dst_pad)
    return _tc_final(p, dis, acc, W1[3], b1, prelu1, Wout, bout)
